# baseline (speedup 1.0000x reference)
import functools

import jax
import jax.numpy as jnp
from jax.experimental import pallas as pl

DEPTH = 2
DIM = 64
NTOK = 21
TDIM = 16
MDIM = 16
B = 2
N = 256
IBLK = 32

_SELU_L = 1.0507009873554805
_SELU_A = 1.6732632423543772


def _silu(x):
    return x * (1.0 / (1.0 + jnp.exp(-x)))


def _selu(x):
    return _SELU_L * jnp.where(x > 0, x, _SELU_A * (jnp.exp(x) - 1.0))


def _egnn_body(feats_ref, coors_ref, time_ref, *refs):
    (emb_r, wt1_r, bt1_r, wt2_r, bt2_r, wt3_r, bt3_r) = refs[:7]
    layer_refs = refs[7:7 + 17 * DEPTH]
    x_out_ref, c_out_ref = refs[7 + 17 * DEPTH:]

    emb = emb_r[...]
    for b in range(B):
        fb = feats_ref[b]
        tok_iota = jax.lax.broadcasted_iota(jnp.int32, (N, NTOK), 1)
        onehot = (fb == tok_iota).astype(jnp.float32)
        x = jnp.dot(onehot, emb, preferred_element_type=jnp.float32)

        t = time_ref[b:b + 1]
        t = _selu(jnp.dot(t, wt1_r[...]) + bt1_r[...])
        t = _selu(jnp.dot(t, wt2_r[...]) + bt2_r[...])
        t = jnp.dot(t, wt3_r[...]) + bt3_r[...]
        x = x + t

        c = coors_ref[b]

        for l in range(DEPTH):
            (w1a_r, w1b_r, w1d_r, b1_r, w2_r, b2_r, wc1_r, bc1_r, wc2_r,
             bc2_r, lng_r, lnb_r, wn1a_r, wn1b_r, bn1_r, wn2_r,
             bn2_r) = layer_refs[17 * l:17 * (l + 1)]

            a_proj = jnp.dot(x, w1a_r[...], preferred_element_type=jnp.float32)
            b_proj = jnp.dot(x, w1b_r[...], preferred_element_type=jnp.float32)
            b_proj = b_proj + b1_r[...]
            w1d = w1d_r[...]

            cT = c.T
            cx = cT[0:1]
            cy = cT[1:2]
            cz = cT[2:3]

            x_blocks = []
            c_blocks = []
            for ib in range(N // IBLK):
                s = ib * IBLK
                ci = c[s:s + IBLK]
                cix = ci[:, 0:1]
                ciy = ci[:, 1:2]
                ciz = ci[:, 2:3]
                relx = cix - cx
                rely = ciy - cy
                relz = ciz - cz
                dist = relx * relx + rely * rely + relz * relz

                pre = (a_proj[s:s + IBLK][:, None, :]
                       + b_proj[None, :, :]
                       + dist[:, :, None] * w1d[None, :, :])
                h = _silu(pre).reshape(IBLK * N, 2 * (2 * DIM + 1))
                m = _silu(jnp.dot(h, w2_r[...],
                                  preferred_element_type=jnp.float32)
                          + b2_r[...])
                cwh = _silu(jnp.dot(m, wc1_r[...],
                                    preferred_element_type=jnp.float32)
                            + bc1_r[...])
                cw = (jnp.dot(cwh, wc2_r[...],
                              preferred_element_type=jnp.float32)
                      + bc2_r[...])
                cw2 = cw.reshape(IBLK, N)

                m3 = m.reshape(IBLK, N, MDIM)
                m_i = jnp.sum(m3, axis=1)

                dx = jnp.sum(cw2 * relx, axis=1, keepdims=True)
                dy = jnp.sum(cw2 * rely, axis=1, keepdims=True)
                dz = jnp.sum(cw2 * relz, axis=1, keepdims=True)
                c_blocks.append(ci + jnp.concatenate([dx, dy, dz], axis=1))

                xi = x[s:s + IBLK]
                mu = jnp.mean(xi, axis=-1, keepdims=True)
                var = jnp.mean((xi - mu) ** 2, axis=-1, keepdims=True)
                normed = (xi - mu) / jnp.sqrt(var + 1e-5) * lng_r[...] \
                    + lnb_r[...]
                h2 = _silu(jnp.dot(normed, wn1a_r[...],
                                   preferred_element_type=jnp.float32)
                           + jnp.dot(m_i, wn1b_r[...],
                                     preferred_element_type=jnp.float32)
                           + bn1_r[...])
                x_blocks.append(jnp.dot(h2, wn2_r[...],
                                        preferred_element_type=jnp.float32)
                                + bn2_r[...] + xi)

            x = jnp.concatenate(x_blocks, axis=0)
            c = jnp.concatenate(c_blocks, axis=0)

        x_out_ref[b] = x
        c_out_ref[b] = c


@jax.jit
def kernel(feats, coors, time, params):
    feats_i = feats.astype(jnp.int32).reshape(B, N, 1)
    coors_f = coors.astype(jnp.float32)
    time_f = time.astype(jnp.float32).reshape(B, 1)

    def lin(p):
        W, bb = p
        return W, bb.reshape(1, -1)

    args = [feats_i, coors_f, time_f, params['token_emb']]
    for name in ('t1', 't2', 't3'):
        W, bb = lin(params[name])
        args += [W, bb]
    for lp in params['layers']:
        W1, b1 = lin(lp['e1'])
        w1a, w1b, w1d = W1[:DIM], W1[DIM:2 * DIM], W1[2 * DIM:2 * DIM + 1]
        W2, b2 = lin(lp['e2'])
        Wc1, bc1 = lin(lp['c1'])
        Wc2, bc2 = lin(lp['c2'])
        Wn1, bn1 = lin(lp['n1'])
        wn1a, wn1b = Wn1[:DIM], Wn1[DIM:]
        Wn2, bn2 = lin(lp['n2'])
        args += [w1a, w1b, w1d, b1, W2, b2, Wc1, bc1, Wc2, bc2,
                 lp['ln_g'].reshape(1, DIM), lp['ln_b'].reshape(1, DIM),
                 wn1a, wn1b, bn1, Wn2, bn2]

    out_shape = (jax.ShapeDtypeStruct((B, N, DIM), jnp.float32),
                 jax.ShapeDtypeStruct((B, N, 3), jnp.float32))
    x_out, c_out = pl.pallas_call(
        _egnn_body,
        out_shape=out_shape,
    )(*args)
    return (x_out, c_out)
